# baseline (speedup 1.0000x reference)
import functools

import jax
import jax.numpy as jnp
from jax import lax
from jax.experimental import pallas as pl
from jax.experimental.pallas import tpu as pltpu
from jax.experimental.pallas import tpu_sc as plsc

VOCAB = 1000000
EMBED = 64
B = 16384
L = 50
NUM_CLASSES = 20

NC = 2
NS = 16
NW = NC * NS

TCHUNK = 128
NFULL = VOCAB // TCHUNK
REM = VOCAB - NFULL * TCHUNK
JMAX = (NFULL + NW - 1) // NW

BAGS_PER_W = B // NW
IDX_PER_W = BAGS_PER_W * L
GROUP_BAGS = 4
GROUP_IDX = GROUP_BAGS * L
NGROUPS = BAGS_PER_W // GROUP_BAGS
CHUNK0 = 104
CHUNK1 = GROUP_IDX - CHUNK0
VREGS = EMBED // 16

_MESH = dict(core_axis_name="c", subcore_axis_name="s",
             num_cores=NC, num_subcores=NS)


def _tr_body(tbl_t, out_ref, slab0, slab1, stg0, stg1, slab_r, stg_r,
             rs0, rs1, ws0, ws1):
    wid = lax.axis_index("s") * NC + lax.axis_index("c")
    slabs, stgs = (slab0, slab1), (stg0, stg1)
    rsems, wsems = (rs0, rs1), (ws0, ws1)
    iotas = tuple(lax.iota(jnp.int32, 16) + 16 * k for k in range(VREGS))

    def chunk_of(j):
        return j * NW + wid

    def rd_descr(j, slot):
        off = pl.multiple_of(chunk_of(j) * TCHUNK, TCHUNK)
        return pltpu.make_async_copy(
            tbl_t.at[:, pl.ds(off, TCHUNK)], slabs[slot], rsems[slot])

    HW = EMBED // 2

    def wr_descr(j, slot):
        off = pl.multiple_of(chunk_of(j) * (TCHUNK * HW), TCHUNK * HW)
        return pltpu.make_async_copy(
            stgs[slot], out_ref.at[pl.ds(off, TCHUNK * HW)], wsems[slot])

    def transpose_into(slab, stg, nrows):
        @plsc.parallel_loop(0, nrows, step=1, unroll=8)
        def _(v):
            col = jnp.full((16,), 0, jnp.int32) + v
            base = pl.multiple_of(v * HW, 16)
            regs = [plsc.load_gather(slab, [iotas[k], col])
                    for k in range(VREGS)]
            lo = plsc.pack(regs[0], regs[1],
                           format=plsc.PackFormat.INTERLEAVED)
            hi = plsc.pack(regs[2], regs[3],
                           format=plsc.PackFormat.INTERLEAVED)
            stg[pl.ds(base, 16)] = plsc.bitcast(lo, jnp.int32)
            stg[pl.ds(base + 16, 16)] = plsc.bitcast(hi, jnp.int32)

    rd_descr(0, 0).start()

    def outer(j2, carry):
        for b2 in range(2):
            j = j2 * 2 + b2

            @pl.when(chunk_of(j + 1) < NFULL)
            def _():
                rd_descr(j + 1, 1 - b2).start()

            @pl.when(chunk_of(j) < NFULL)
            def _():
                rd_descr(j, b2).wait()

                @pl.when(j >= 2)
                def _():
                    wr_descr(j - 2, b2).wait()

                transpose_into(slabs[b2], stgs[b2], TCHUNK)
                wr_descr(j, b2).start()
        return carry

    lax.fori_loop(0, (JMAX + 1) // 2, outer, 0)

    for j in (JMAX - 3, JMAX - 2, JMAX - 1):
        @pl.when((chunk_of(j) < NFULL) & (chunk_of(j + 2) >= NFULL))
        def _():
            wr_descr(j, j % 2).wait()

    @pl.when(wid == NW - 1)
    def _():
        pltpu.sync_copy(tbl_t.at[:, pl.ds(VOCAB - REM, REM)], slab_r)
        transpose_into(slab_r, stg_r, REM)
        pltpu.sync_copy(
            stg_r,
            out_ref.at[pl.ds((VOCAB - REM) * (EMBED // 2),
                             REM * (EMBED // 2))])


def _sc_body(data_ref, table_ref, out_ref, idx_v, rows0, rows1, pooled_v,
             sem0, sem1):
    wid = lax.axis_index("s") * NC + lax.axis_index("c")
    pltpu.sync_copy(data_ref.at[pl.ds(wid * IDX_PER_W, IDX_PER_W)], idx_v)

    bufs = (rows0, rows1)
    sems = (sem0, sem1)

    def gather_descrs(g, slot):
        off = pl.multiple_of(g * GROUP_IDX, 8)
        buf, sem = bufs[slot], sems[slot]
        return (
            pltpu.make_async_copy(
                table_ref.at[idx_v.at[pl.ds(off, CHUNK0)]],
                buf.at[pl.ds(0, CHUNK0)], sem),
            pltpu.make_async_copy(
                table_ref.at[idx_v.at[pl.ds(off + CHUNK0, CHUNK1)]],
                buf.at[pl.ds(CHUNK0, CHUNK1)], sem),
        )

    def issue(g, slot):
        for d in gather_descrs(g, slot):
            d.start()

    def drain(g, slot):
        for d in gather_descrs(g, slot):
            d.wait()

    def compute(g, slot):
        buf = bufs[slot]
        for bag in range(GROUP_BAGS):
            def rbody(r, acc):
                row = bag * L + r
                a0, b0 = plsc.unpack(buf[row, pl.ds(0, 32)],
                                     format=plsc.PackFormat.INTERLEAVED)
                a1, b1 = plsc.unpack(buf[row, pl.ds(32, 32)],
                                     format=plsc.PackFormat.INTERLEAVED)
                return (acc[0] + a0, acc[1] + b0, acc[2] + a1, acc[3] + b1)
            acc = lax.fori_loop(
                0, L, rbody,
                tuple(jnp.zeros((16,), jnp.float32) for _ in range(VREGS)))
            for k in range(VREGS):
                pooled_v[g * GROUP_BAGS + bag, pl.ds(k * 16, 16)] = acc[k]

    issue(0, 0)

    def outer(g2, carry):
        for b in range(2):
            g = g2 * 2 + b

            @pl.when(g + 1 < NGROUPS)
            def _():
                issue(g + 1, 1 - b)

            drain(g, b)
            compute(g, b)
        return carry

    lax.fori_loop(0, NGROUPS // 2, outer, 0)

    pltpu.sync_copy(pooled_v, out_ref.at[pl.ds(wid * BAGS_PER_W, BAGS_PER_W)])


def _tc_head(pooled_ref, w_ref, b_ref, out_ref):
    out_ref[...] = (
        jnp.dot(pooled_ref[...], w_ref[...],
                preferred_element_type=jnp.float32)
        + b_ref[...]
    )


@jax.jit
def kernel(data, table, W, b):
    data_flat = data.reshape(-1)

    sc_transpose = pl.kernel(
        _tr_body,
        out_type=jax.ShapeDtypeStruct((VOCAB * EMBED // 2,), jnp.int32),
        mesh=plsc.VectorSubcoreMesh(**_MESH),
        scratch_types=[
            pltpu.VMEM((EMBED, TCHUNK), jnp.float32),
            pltpu.VMEM((EMBED, TCHUNK), jnp.float32),
            pltpu.VMEM((TCHUNK * EMBED // 2,), jnp.int32),
            pltpu.VMEM((TCHUNK * EMBED // 2,), jnp.int32),
            pltpu.VMEM((EMBED, REM), jnp.float32),
            pltpu.VMEM((REM * EMBED // 2,), jnp.int32),
            pltpu.SemaphoreType.DMA,
            pltpu.SemaphoreType.DMA,
            pltpu.SemaphoreType.DMA,
            pltpu.SemaphoreType.DMA,
        ],
        compiler_params=pltpu.CompilerParams(
            use_tc_tiling_on_sc=True, needs_layout_passes=False),
    )
    table_rows = lax.bitcast_convert_type(
        sc_transpose(table.T), jnp.bfloat16).reshape(VOCAB, EMBED)

    sc_pool = pl.kernel(
        _sc_body,
        out_type=jax.ShapeDtypeStruct((B, EMBED), jnp.float32),
        mesh=plsc.VectorSubcoreMesh(**_MESH),
        scratch_types=[
            pltpu.VMEM((IDX_PER_W,), jnp.int32),
            pltpu.VMEM((GROUP_IDX, EMBED), jnp.bfloat16),
            pltpu.VMEM((GROUP_IDX, EMBED), jnp.bfloat16),
            pltpu.VMEM((BAGS_PER_W, EMBED), jnp.float32),
            pltpu.SemaphoreType.DMA,
            pltpu.SemaphoreType.DMA,
        ],
        compiler_params=pltpu.CompilerParams(
            use_tc_tiling_on_sc=False, needs_layout_passes=False),
    )
    pooled_sum = sc_pool(data_flat, table_rows)

    w_pad = jnp.zeros((EMBED, 128), jnp.float32)
    w_pad = lax.dynamic_update_slice(w_pad, W.T * (1.0 / L), (0, 0))
    b_pad = jnp.zeros((1, 128), jnp.float32)
    b_pad = lax.dynamic_update_slice(b_pad, b[None, :], (0, 0))

    blk = 2048
    logits_pad = pl.pallas_call(
        _tc_head,
        grid=(B // blk,),
        in_specs=[
            pl.BlockSpec((blk, EMBED), lambda i: (i, 0)),
            pl.BlockSpec((EMBED, 128), lambda i: (0, 0)),
            pl.BlockSpec((1, 128), lambda i: (0, 0)),
        ],
        out_specs=pl.BlockSpec((blk, 128), lambda i: (i, 0)),
        out_shape=jax.ShapeDtypeStruct((B, 128), jnp.float32),
    )(pooled_sum, w_pad, b_pad)

    return logits_pad[:, :NUM_CLASSES]


# speedup vs baseline: 2.8190x; 2.8190x over previous
import functools

import jax
import jax.numpy as jnp
from jax import lax
from jax.experimental import pallas as pl
from jax.experimental.pallas import tpu as pltpu
from jax.experimental.pallas import tpu_sc as plsc

VOCAB = 1000000
EMBED = 64
B = 16384
L = 50
NUM_CLASSES = 20

NC = 2
NS = 16
NW = NC * NS

TCHUNK = 128
NFULL = VOCAB // TCHUNK
REM = VOCAB - NFULL * TCHUNK
JMAX = (NFULL + NW - 1) // NW

BAGS_PER_W = B // NW
IDX_PER_W = BAGS_PER_W * L
GROUP_BAGS = 4
GROUP_IDX = GROUP_BAGS * L
NGROUPS = BAGS_PER_W // GROUP_BAGS
CHUNK0 = 104
CHUNK1 = GROUP_IDX - CHUNK0
VREGS = EMBED // 16

_MESH = dict(core_axis_name="c", subcore_axis_name="s",
             num_cores=NC, num_subcores=NS)


def _tr_body(tbl_t, out_ref, slab0, slab1, stg0, stg1, slab_r, stg_r,
             rs0, rs1, ws0, ws1):
    wid = lax.axis_index("s") * NC + lax.axis_index("c")
    slabs, stgs = (slab0, slab1), (stg0, stg1)
    rsems, wsems = (rs0, rs1), (ws0, ws1)
    iotas = tuple(lax.iota(jnp.int32, 16) + 16 * k for k in range(VREGS))

    def chunk_of(j):
        return j * NW + wid

    def rd_descr(j, slot):
        off = pl.multiple_of(chunk_of(j) * TCHUNK, TCHUNK)
        return pltpu.make_async_copy(
            tbl_t.at[:, pl.ds(off, TCHUNK)], slabs[slot], rsems[slot])

    HW = EMBED // 2

    def wr_descr(j, slot):
        off = pl.multiple_of(chunk_of(j) * (TCHUNK * HW), TCHUNK * HW)
        return pltpu.make_async_copy(
            stgs[slot], out_ref.at[pl.ds(off, TCHUNK * HW)], wsems[slot])

    def transpose_into(slab, stg, nrows):
        @plsc.parallel_loop(0, nrows, step=1, unroll=8,
                            carry=jnp.zeros((16,), jnp.int32))
        def _(v, col):
            base = pl.multiple_of(v * HW, 16)
            regs = [plsc.load_gather(slab, [iotas[k], col])
                    for k in range(VREGS)]
            lo = plsc.pack(regs[0], regs[1],
                           format=plsc.PackFormat.INTERLEAVED)
            hi = plsc.pack(regs[2], regs[3],
                           format=plsc.PackFormat.INTERLEAVED)
            stg[pl.ds(base, 16)] = plsc.bitcast(lo, jnp.int32)
            stg[pl.ds(base + 16, 16)] = plsc.bitcast(hi, jnp.int32)
            return col + 1

    rd_descr(0, 0).start()

    def outer(j2, carry):
        for b2 in range(2):
            j = j2 * 2 + b2

            @pl.when(chunk_of(j + 1) < NFULL)
            def _():
                rd_descr(j + 1, 1 - b2).start()

            @pl.when(chunk_of(j) < NFULL)
            def _():
                rd_descr(j, b2).wait()

                @pl.when(j >= 2)
                def _():
                    wr_descr(j - 2, b2).wait()

                transpose_into(slabs[b2], stgs[b2], TCHUNK)
                wr_descr(j, b2).start()
        return carry

    lax.fori_loop(0, (JMAX + 1) // 2, outer, 0)

    for j in (JMAX - 3, JMAX - 2, JMAX - 1):
        @pl.when((chunk_of(j) < NFULL) & (chunk_of(j + 2) >= NFULL))
        def _():
            wr_descr(j, j % 2).wait()

    @pl.when(wid == NW - 1)
    def _():
        pltpu.sync_copy(tbl_t.at[:, pl.ds(VOCAB - REM, REM)], slab_r)
        transpose_into(slab_r, stg_r, REM)
        pltpu.sync_copy(
            stg_r,
            out_ref.at[pl.ds((VOCAB - REM) * (EMBED // 2),
                             REM * (EMBED // 2))])


def _sc_body(data_ref, table_ref, out_ref, idx_v, rows0, rows1, pooled_v,
             sem0, sem1):
    wid = lax.axis_index("s") * NC + lax.axis_index("c")
    pltpu.sync_copy(data_ref.at[pl.ds(wid * IDX_PER_W, IDX_PER_W)], idx_v)

    bufs = (rows0, rows1)
    sems = (sem0, sem1)

    def gather_descrs(g, slot):
        off = pl.multiple_of(g * GROUP_IDX, 8)
        buf, sem = bufs[slot], sems[slot]
        return (
            pltpu.make_async_copy(
                table_ref.at[idx_v.at[pl.ds(off, CHUNK0)]],
                buf.at[pl.ds(0, CHUNK0)], sem),
            pltpu.make_async_copy(
                table_ref.at[idx_v.at[pl.ds(off + CHUNK0, CHUNK1)]],
                buf.at[pl.ds(CHUNK0, CHUNK1)], sem),
        )

    def issue(g, slot):
        for d in gather_descrs(g, slot):
            d.start()

    def drain(g, slot):
        for d in gather_descrs(g, slot):
            d.wait()

    def compute(g, slot):
        buf = bufs[slot]
        for bag in range(GROUP_BAGS):
            def rbody(r, acc):
                row = bag * L + r
                a0, b0 = plsc.unpack(
                    plsc.bitcast(buf[row, pl.ds(0, 16)], jnp.bfloat16),
                    format=plsc.PackFormat.INTERLEAVED)
                a1, b1 = plsc.unpack(
                    plsc.bitcast(buf[row, pl.ds(16, 16)], jnp.bfloat16),
                    format=plsc.PackFormat.INTERLEAVED)
                return (acc[0] + a0, acc[1] + b0, acc[2] + a1, acc[3] + b1)
            acc = lax.fori_loop(
                0, L, rbody,
                tuple(jnp.zeros((16,), jnp.float32) for _ in range(VREGS)))
            for k in range(VREGS):
                pooled_v[g * GROUP_BAGS + bag, pl.ds(k * 16, 16)] = acc[k]

    issue(0, 0)

    def outer(g2, carry):
        for b in range(2):
            g = g2 * 2 + b

            @pl.when(g + 1 < NGROUPS)
            def _():
                issue(g + 1, 1 - b)

            drain(g, b)
            compute(g, b)
        return carry

    lax.fori_loop(0, NGROUPS // 2, outer, 0)

    pltpu.sync_copy(pooled_v, out_ref.at[pl.ds(wid * BAGS_PER_W, BAGS_PER_W)])


def _tc_head(pooled_ref, w_ref, b_ref, out_ref):
    out_ref[...] = (
        jnp.dot(pooled_ref[...], w_ref[...],
                preferred_element_type=jnp.float32)
        + b_ref[...]
    )


@jax.jit
def kernel(data, table, W, b):
    data_flat = data.reshape(-1)

    sc_transpose = pl.kernel(
        _tr_body,
        out_type=jax.ShapeDtypeStruct((VOCAB * EMBED // 2,), jnp.int32),
        mesh=plsc.VectorSubcoreMesh(**_MESH),
        scratch_types=[
            pltpu.VMEM((EMBED, TCHUNK), jnp.float32),
            pltpu.VMEM((EMBED, TCHUNK), jnp.float32),
            pltpu.VMEM((TCHUNK * EMBED // 2,), jnp.int32),
            pltpu.VMEM((TCHUNK * EMBED // 2,), jnp.int32),
            pltpu.VMEM((EMBED, REM), jnp.float32),
            pltpu.VMEM((REM * EMBED // 2,), jnp.int32),
            pltpu.SemaphoreType.DMA,
            pltpu.SemaphoreType.DMA,
            pltpu.SemaphoreType.DMA,
            pltpu.SemaphoreType.DMA,
        ],
        compiler_params=pltpu.CompilerParams(
            use_tc_tiling_on_sc=True, needs_layout_passes=False),
    )
    table_rows = sc_transpose(table.T).reshape(VOCAB, EMBED // 2)

    sc_pool = pl.kernel(
        _sc_body,
        out_type=jax.ShapeDtypeStruct((B, EMBED), jnp.float32),
        mesh=plsc.VectorSubcoreMesh(**_MESH),
        scratch_types=[
            pltpu.VMEM((IDX_PER_W,), jnp.int32),
            pltpu.VMEM((GROUP_IDX, EMBED // 2), jnp.int32),
            pltpu.VMEM((GROUP_IDX, EMBED // 2), jnp.int32),
            pltpu.VMEM((BAGS_PER_W, EMBED), jnp.float32),
            pltpu.SemaphoreType.DMA,
            pltpu.SemaphoreType.DMA,
        ],
        compiler_params=pltpu.CompilerParams(
            use_tc_tiling_on_sc=False, needs_layout_passes=False),
    )
    pooled_sum = sc_pool(data_flat, table_rows)

    w_pad = jnp.zeros((EMBED, 128), jnp.float32)
    w_pad = lax.dynamic_update_slice(w_pad, W.T * (1.0 / L), (0, 0))
    b_pad = jnp.zeros((1, 128), jnp.float32)
    b_pad = lax.dynamic_update_slice(b_pad, b[None, :], (0, 0))

    blk = 2048
    logits_pad = pl.pallas_call(
        _tc_head,
        grid=(B // blk,),
        in_specs=[
            pl.BlockSpec((blk, EMBED), lambda i: (i, 0)),
            pl.BlockSpec((EMBED, 128), lambda i: (0, 0)),
            pl.BlockSpec((1, 128), lambda i: (0, 0)),
        ],
        out_specs=pl.BlockSpec((blk, 128), lambda i: (i, 0)),
        out_shape=jax.ShapeDtypeStruct((B, 128), jnp.float32),
    )(pooled_sum, w_pad, b_pad)

    return logits_pad[:, :NUM_CLASSES]


# speedup vs baseline: 2.8396x; 1.0073x over previous
import functools

import jax
import jax.numpy as jnp
from jax import lax
from jax.experimental import pallas as pl
from jax.experimental.pallas import tpu as pltpu
from jax.experimental.pallas import tpu_sc as plsc

VOCAB = 1000000
EMBED = 64
B = 16384
L = 50
NUM_CLASSES = 20

NC = 2
NS = 16
NW = NC * NS

TCHUNK = 128
NFULL = VOCAB // TCHUNK
REM = VOCAB - NFULL * TCHUNK
JMAX = (NFULL + NW - 1) // NW

BAGS_PER_W = B // NW
IDX_PER_W = BAGS_PER_W * L
GROUP_BAGS = 4
GROUP_IDX = GROUP_BAGS * L
NGROUPS = BAGS_PER_W // GROUP_BAGS
CHUNK0 = 104
CHUNK1 = GROUP_IDX - CHUNK0
VREGS = EMBED // 16

_MESH = dict(core_axis_name="c", subcore_axis_name="s",
             num_cores=NC, num_subcores=NS)


def _tr_body(tbl_t, out_ref, slab0, slab1, stg0, stg1, slab_r, stg_r,
             rs0, rs1, ws0, ws1):
    wid = lax.axis_index("s") * NC + lax.axis_index("c")
    slabs, stgs = (slab0, slab1), (stg0, stg1)
    rsems, wsems = (rs0, rs1), (ws0, ws1)
    iotas = tuple(lax.iota(jnp.int32, 16) + 16 * k for k in range(VREGS))

    def chunk_of(j):
        return j * NW + wid

    def rd_descr(j, slot):
        off = pl.multiple_of(chunk_of(j) * TCHUNK, TCHUNK)
        return pltpu.make_async_copy(
            tbl_t.at[:, pl.ds(off, TCHUNK)],
            slabs[slot].at[:, pl.ds(0, TCHUNK)], rsems[slot])

    HW = EMBED // 2

    def wr_descr(j, slot):
        off = pl.multiple_of(chunk_of(j) * (TCHUNK * HW), TCHUNK * HW)
        return pltpu.make_async_copy(
            stgs[slot], out_ref.at[pl.ds(off, TCHUNK * HW)], wsems[slot])

    def transpose_into(slab, stg, nrows):
        @plsc.parallel_loop(0, nrows, step=1, unroll=8,
                            carry=jnp.zeros((16,), jnp.int32))
        def _(v, col):
            base = pl.multiple_of(v * HW, 16)
            regs = [plsc.load_gather(slab, [iotas[k], col])
                    for k in range(VREGS)]
            lo = plsc.pack(regs[0], regs[1],
                           format=plsc.PackFormat.INTERLEAVED)
            hi = plsc.pack(regs[2], regs[3],
                           format=plsc.PackFormat.INTERLEAVED)
            stg[pl.ds(base, 16)] = plsc.bitcast(lo, jnp.int32)
            stg[pl.ds(base + 16, 16)] = plsc.bitcast(hi, jnp.int32)
            return col + 1

    rd_descr(0, 0).start()

    def outer(j2, carry):
        for b2 in range(2):
            j = j2 * 2 + b2

            @pl.when(chunk_of(j + 1) < NFULL)
            def _():
                rd_descr(j + 1, 1 - b2).start()

            @pl.when(chunk_of(j) < NFULL)
            def _():
                rd_descr(j, b2).wait()

                @pl.when(j >= 2)
                def _():
                    wr_descr(j - 2, b2).wait()

                transpose_into(slabs[b2], stgs[b2], TCHUNK)
                wr_descr(j, b2).start()
        return carry

    lax.fori_loop(0, (JMAX + 1) // 2, outer, 0)

    for j in (JMAX - 3, JMAX - 2, JMAX - 1):
        @pl.when((chunk_of(j) < NFULL) & (chunk_of(j + 2) >= NFULL))
        def _():
            wr_descr(j, j % 2).wait()

    @pl.when(wid == NW - 1)
    def _():
        pltpu.sync_copy(tbl_t.at[:, pl.ds(VOCAB - REM, REM)], slab_r)
        transpose_into(slab_r, stg_r, REM)
        pltpu.sync_copy(
            stg_r,
            out_ref.at[pl.ds((VOCAB - REM) * (EMBED // 2),
                             REM * (EMBED // 2))])


def _sc_body(data_ref, table_ref, out_ref, idx_v, rows0, rows1, pooled_v,
             sem0, sem1):
    wid = lax.axis_index("s") * NC + lax.axis_index("c")
    pltpu.sync_copy(data_ref.at[pl.ds(wid * IDX_PER_W, IDX_PER_W)], idx_v)

    bufs = (rows0, rows1)
    sems = (sem0, sem1)

    def gather_descrs(g, slot):
        off = pl.multiple_of(g * GROUP_IDX, 8)
        buf, sem = bufs[slot], sems[slot]
        return (
            pltpu.make_async_copy(
                table_ref.at[idx_v.at[pl.ds(off, CHUNK0)]],
                buf.at[pl.ds(0, CHUNK0)], sem),
            pltpu.make_async_copy(
                table_ref.at[idx_v.at[pl.ds(off + CHUNK0, CHUNK1)]],
                buf.at[pl.ds(CHUNK0, CHUNK1)], sem),
        )

    def issue(g, slot):
        for d in gather_descrs(g, slot):
            d.start()

    def drain(g, slot):
        for d in gather_descrs(g, slot):
            d.wait()

    def compute(g, slot):
        buf = bufs[slot]
        for bag in range(GROUP_BAGS):
            def rbody(r, acc):
                row = bag * L + r
                a0, b0 = plsc.unpack(
                    plsc.bitcast(buf[row, pl.ds(0, 16)], jnp.bfloat16),
                    format=plsc.PackFormat.INTERLEAVED)
                a1, b1 = plsc.unpack(
                    plsc.bitcast(buf[row, pl.ds(16, 16)], jnp.bfloat16),
                    format=plsc.PackFormat.INTERLEAVED)
                return (acc[0] + a0, acc[1] + b0, acc[2] + a1, acc[3] + b1)
            acc = lax.fori_loop(
                0, L, rbody,
                tuple(jnp.zeros((16,), jnp.float32) for _ in range(VREGS)))
            for k in range(VREGS):
                pooled_v[g * GROUP_BAGS + bag, pl.ds(k * 16, 16)] = acc[k]

    issue(0, 0)

    def outer(g2, carry):
        for b in range(2):
            g = g2 * 2 + b

            @pl.when(g + 1 < NGROUPS)
            def _():
                issue(g + 1, 1 - b)

            drain(g, b)
            compute(g, b)
        return carry

    lax.fori_loop(0, NGROUPS // 2, outer, 0)

    pltpu.sync_copy(pooled_v, out_ref.at[pl.ds(wid * BAGS_PER_W, BAGS_PER_W)])


def _tc_head(pooled_ref, w_ref, b_ref, out_ref):
    out_ref[...] = (
        jnp.dot(pooled_ref[...], w_ref[...],
                preferred_element_type=jnp.float32)
        + b_ref[...]
    )


@jax.jit
def kernel(data, table, W, b):
    data_flat = data.reshape(-1)

    sc_transpose = pl.kernel(
        _tr_body,
        out_type=jax.ShapeDtypeStruct((VOCAB * EMBED // 2,), jnp.int32),
        mesh=plsc.VectorSubcoreMesh(**_MESH),
        scratch_types=[
            pltpu.VMEM((EMBED, TCHUNK + 1), jnp.float32),
            pltpu.VMEM((EMBED, TCHUNK + 1), jnp.float32),
            pltpu.VMEM((TCHUNK * EMBED // 2,), jnp.int32),
            pltpu.VMEM((TCHUNK * EMBED // 2,), jnp.int32),
            pltpu.VMEM((EMBED, REM), jnp.float32),
            pltpu.VMEM((REM * EMBED // 2,), jnp.int32),
            pltpu.SemaphoreType.DMA,
            pltpu.SemaphoreType.DMA,
            pltpu.SemaphoreType.DMA,
            pltpu.SemaphoreType.DMA,
        ],
        compiler_params=pltpu.CompilerParams(
            use_tc_tiling_on_sc=True, needs_layout_passes=False),
    )
    table_rows = sc_transpose(table.T).reshape(VOCAB, EMBED // 2)

    sc_pool = pl.kernel(
        _sc_body,
        out_type=jax.ShapeDtypeStruct((B, EMBED), jnp.float32),
        mesh=plsc.VectorSubcoreMesh(**_MESH),
        scratch_types=[
            pltpu.VMEM((IDX_PER_W,), jnp.int32),
            pltpu.VMEM((GROUP_IDX, EMBED // 2), jnp.int32),
            pltpu.VMEM((GROUP_IDX, EMBED // 2), jnp.int32),
            pltpu.VMEM((BAGS_PER_W, EMBED), jnp.float32),
            pltpu.SemaphoreType.DMA,
            pltpu.SemaphoreType.DMA,
        ],
        compiler_params=pltpu.CompilerParams(
            use_tc_tiling_on_sc=False, needs_layout_passes=False),
    )
    pooled_sum = sc_pool(data_flat, table_rows)

    w_pad = jnp.zeros((EMBED, 128), jnp.float32)
    w_pad = lax.dynamic_update_slice(w_pad, W.T * (1.0 / L), (0, 0))
    b_pad = jnp.zeros((1, 128), jnp.float32)
    b_pad = lax.dynamic_update_slice(b_pad, b[None, :], (0, 0))

    blk = 2048
    logits_pad = pl.pallas_call(
        _tc_head,
        grid=(B // blk,),
        in_specs=[
            pl.BlockSpec((blk, EMBED), lambda i: (i, 0)),
            pl.BlockSpec((EMBED, 128), lambda i: (0, 0)),
            pl.BlockSpec((1, 128), lambda i: (0, 0)),
        ],
        out_specs=pl.BlockSpec((blk, 128), lambda i: (i, 0)),
        out_shape=jax.ShapeDtypeStruct((B, 128), jnp.float32),
    )(pooled_sum, w_pad, b_pad)

    return logits_pad[:, :NUM_CLASSES]


# speedup vs baseline: 2.9654x; 1.0443x over previous
import functools

import jax
import jax.numpy as jnp
from jax import lax
from jax.experimental import pallas as pl
from jax.experimental.pallas import tpu as pltpu
from jax.experimental.pallas import tpu_sc as plsc

VOCAB = 1000000
EMBED = 64
B = 16384
L = 50
NUM_CLASSES = 20

NC = 2
NS = 16
NW = NC * NS

TCHUNK = 512
NFULL = VOCAB // TCHUNK
REM = VOCAB - NFULL * TCHUNK
JMAX = (NFULL + NW - 1) // NW

BAGS_PER_W = B // NW
IDX_PER_W = BAGS_PER_W * L
GROUP_BAGS = 4
GROUP_IDX = GROUP_BAGS * L
NGROUPS = BAGS_PER_W // GROUP_BAGS
CHUNK0 = 104
CHUNK1 = GROUP_IDX - CHUNK0
VREGS = EMBED // 16

_MESH = dict(core_axis_name="c", subcore_axis_name="s",
             num_cores=NC, num_subcores=NS)


def _tr_body(tbl_t, out_ref, slab0, slab1, stg0, stg1, slab_r, stg_r,
             rs0, rs1, ws0, ws1):
    wid = lax.axis_index("s") * NC + lax.axis_index("c")
    slabs, stgs = (slab0, slab1), (stg0, stg1)
    rsems, wsems = (rs0, rs1), (ws0, ws1)
    iotas = tuple(lax.iota(jnp.int32, 16) + 16 * k for k in range(VREGS))

    def chunk_of(j):
        return j * NW + wid

    def rd_descr(j, slot):
        off = pl.multiple_of(chunk_of(j) * TCHUNK, TCHUNK)
        return pltpu.make_async_copy(
            tbl_t.at[:, pl.ds(off, TCHUNK)],
            slabs[slot].at[:, pl.ds(0, TCHUNK)], rsems[slot])

    HW = EMBED // 2

    def wr_descr(j, slot):
        off = pl.multiple_of(chunk_of(j) * (TCHUNK * HW), TCHUNK * HW)
        return pltpu.make_async_copy(
            stgs[slot], out_ref.at[pl.ds(off, TCHUNK * HW)], wsems[slot])

    def transpose_into(slab, stg, nrows):
        @plsc.parallel_loop(0, nrows, step=1, unroll=16,
                            carry=jnp.zeros((16,), jnp.int32))
        def _(v, col):
            base = pl.multiple_of(v * HW, 16)
            regs = [plsc.load_gather(slab, [iotas[k], col])
                    for k in range(VREGS)]
            lo = plsc.pack(regs[0], regs[1],
                           format=plsc.PackFormat.INTERLEAVED)
            hi = plsc.pack(regs[2], regs[3],
                           format=plsc.PackFormat.INTERLEAVED)
            stg[pl.ds(base, 16)] = plsc.bitcast(lo, jnp.int32)
            stg[pl.ds(base + 16, 16)] = plsc.bitcast(hi, jnp.int32)
            return col + 1

    rd_descr(0, 0).start()

    def outer(j2, carry):
        for b2 in range(2):
            j = j2 * 2 + b2

            @pl.when(chunk_of(j + 1) < NFULL)
            def _():
                rd_descr(j + 1, 1 - b2).start()

            @pl.when(chunk_of(j) < NFULL)
            def _():
                rd_descr(j, b2).wait()

                @pl.when(j >= 2)
                def _():
                    wr_descr(j - 2, b2).wait()

                transpose_into(slabs[b2], stgs[b2], TCHUNK)
                wr_descr(j, b2).start()
        return carry

    lax.fori_loop(0, (JMAX + 1) // 2, outer, 0)

    for j in (JMAX - 3, JMAX - 2, JMAX - 1):
        @pl.when((chunk_of(j) < NFULL) & (chunk_of(j + 2) >= NFULL))
        def _():
            wr_descr(j, j % 2).wait()

    @pl.when(wid == NW - 1)
    def _():
        pltpu.sync_copy(tbl_t.at[:, pl.ds(VOCAB - REM, REM)], slab_r)
        transpose_into(slab_r, stg_r, REM)
        pltpu.sync_copy(
            stg_r,
            out_ref.at[pl.ds((VOCAB - REM) * (EMBED // 2),
                             REM * (EMBED // 2))])


def _sc_body(data_ref, table_ref, out_ref, idx_v, rows0, rows1, pooled_v,
             sem0, sem1):
    wid = lax.axis_index("s") * NC + lax.axis_index("c")
    pltpu.sync_copy(data_ref.at[pl.ds(wid * IDX_PER_W, IDX_PER_W)], idx_v)

    bufs = (rows0, rows1)
    sems = (sem0, sem1)

    def gather_descrs(g, slot):
        off = pl.multiple_of(g * GROUP_IDX, 8)
        buf, sem = bufs[slot], sems[slot]
        return (
            pltpu.make_async_copy(
                table_ref.at[idx_v.at[pl.ds(off, CHUNK0)]],
                buf.at[pl.ds(0, CHUNK0)], sem),
            pltpu.make_async_copy(
                table_ref.at[idx_v.at[pl.ds(off + CHUNK0, CHUNK1)]],
                buf.at[pl.ds(CHUNK0, CHUNK1)], sem),
        )

    def issue(g, slot):
        for d in gather_descrs(g, slot):
            d.start()

    def drain(g, slot):
        for d in gather_descrs(g, slot):
            d.wait()

    def compute(g, slot):
        buf = bufs[slot]
        for bag in range(GROUP_BAGS):
            def rbody(r, acc):
                row = bag * L + r
                a0, b0 = plsc.unpack(
                    plsc.bitcast(buf[row, pl.ds(0, 16)], jnp.bfloat16),
                    format=plsc.PackFormat.INTERLEAVED)
                a1, b1 = plsc.unpack(
                    plsc.bitcast(buf[row, pl.ds(16, 16)], jnp.bfloat16),
                    format=plsc.PackFormat.INTERLEAVED)
                return (acc[0] + a0, acc[1] + b0, acc[2] + a1, acc[3] + b1)
            acc = lax.fori_loop(
                0, L, rbody,
                tuple(jnp.zeros((16,), jnp.float32) for _ in range(VREGS)))
            for k in range(VREGS):
                pooled_v[g * GROUP_BAGS + bag, pl.ds(k * 16, 16)] = acc[k]

    issue(0, 0)

    def outer(g2, carry):
        for b in range(2):
            g = g2 * 2 + b

            @pl.when(g + 1 < NGROUPS)
            def _():
                issue(g + 1, 1 - b)

            drain(g, b)
            compute(g, b)
        return carry

    lax.fori_loop(0, NGROUPS // 2, outer, 0)

    pltpu.sync_copy(pooled_v, out_ref.at[pl.ds(wid * BAGS_PER_W, BAGS_PER_W)])


def _tc_head(pooled_ref, w_ref, b_ref, out_ref):
    out_ref[...] = (
        jnp.dot(pooled_ref[...], w_ref[...],
                preferred_element_type=jnp.float32)
        + b_ref[...]
    )


@jax.jit
def kernel(data, table, W, b):
    data_flat = data.reshape(-1)

    sc_transpose = pl.kernel(
        _tr_body,
        out_type=jax.ShapeDtypeStruct((VOCAB * EMBED // 2,), jnp.int32),
        mesh=plsc.VectorSubcoreMesh(**_MESH),
        scratch_types=[
            pltpu.VMEM((EMBED, TCHUNK + 1), jnp.float32),
            pltpu.VMEM((EMBED, TCHUNK + 1), jnp.float32),
            pltpu.VMEM((TCHUNK * EMBED // 2,), jnp.int32),
            pltpu.VMEM((TCHUNK * EMBED // 2,), jnp.int32),
            pltpu.VMEM((EMBED, REM), jnp.float32),
            pltpu.VMEM((REM * EMBED // 2,), jnp.int32),
            pltpu.SemaphoreType.DMA,
            pltpu.SemaphoreType.DMA,
            pltpu.SemaphoreType.DMA,
            pltpu.SemaphoreType.DMA,
        ],
        compiler_params=pltpu.CompilerParams(
            use_tc_tiling_on_sc=True, needs_layout_passes=False),
    )
    table_rows = sc_transpose(table.T).reshape(VOCAB, EMBED // 2)

    sc_pool = pl.kernel(
        _sc_body,
        out_type=jax.ShapeDtypeStruct((B, EMBED), jnp.float32),
        mesh=plsc.VectorSubcoreMesh(**_MESH),
        scratch_types=[
            pltpu.VMEM((IDX_PER_W,), jnp.int32),
            pltpu.VMEM((GROUP_IDX, EMBED // 2), jnp.int32),
            pltpu.VMEM((GROUP_IDX, EMBED // 2), jnp.int32),
            pltpu.VMEM((BAGS_PER_W, EMBED), jnp.float32),
            pltpu.SemaphoreType.DMA,
            pltpu.SemaphoreType.DMA,
        ],
        compiler_params=pltpu.CompilerParams(
            use_tc_tiling_on_sc=False, needs_layout_passes=False),
    )
    pooled_sum = sc_pool(data_flat, table_rows)

    w_pad = jnp.zeros((EMBED, 128), jnp.float32)
    w_pad = lax.dynamic_update_slice(w_pad, W.T * (1.0 / L), (0, 0))
    b_pad = jnp.zeros((1, 128), jnp.float32)
    b_pad = lax.dynamic_update_slice(b_pad, b[None, :], (0, 0))

    blk = 2048
    logits_pad = pl.pallas_call(
        _tc_head,
        grid=(B // blk,),
        in_specs=[
            pl.BlockSpec((blk, EMBED), lambda i: (i, 0)),
            pl.BlockSpec((EMBED, 128), lambda i: (0, 0)),
            pl.BlockSpec((1, 128), lambda i: (0, 0)),
        ],
        out_specs=pl.BlockSpec((blk, 128), lambda i: (i, 0)),
        out_shape=jax.ShapeDtypeStruct((B, 128), jnp.float32),
    )(pooled_sum, w_pad, b_pad)

    return logits_pad[:, :NUM_CLASSES]


# speedup vs baseline: 4.2931x; 1.4477x over previous
import functools

import jax
import jax.numpy as jnp
from jax import lax
from jax.experimental import pallas as pl
from jax.experimental.pallas import tpu as pltpu
from jax.experimental.pallas import tpu_sc as plsc

VOCAB = 1000000
EMBED = 64
B = 16384
L = 50
NUM_CLASSES = 20

NC = 2
NS = 16
NW = NC * NS
BAGS_PER_W = B // NW
IDX_PER_W = BAGS_PER_W * L
GROUP_BAGS = 4
GROUP_IDX = GROUP_BAGS * L
NGROUPS = BAGS_PER_W // GROUP_BAGS
CHUNK0 = 104
CHUNK1 = GROUP_IDX - CHUNK0
VREGS = EMBED // 16


def _sc_body(data_ref, table_ref, out_ref, idx_v, rows0, rows1, pooled_v,
             sem0, sem1):
    wid = lax.axis_index("s") * NC + lax.axis_index("c")
    pltpu.sync_copy(data_ref.at[pl.ds(wid * IDX_PER_W, IDX_PER_W)], idx_v)

    bufs = (rows0, rows1)
    sems = (sem0, sem1)

    def gather_descrs(g, slot):
        off = pl.multiple_of(g * GROUP_IDX, 8)
        buf, sem = bufs[slot], sems[slot]
        return (
            pltpu.make_async_copy(
                table_ref.at[idx_v.at[pl.ds(off, CHUNK0)]],
                buf.at[pl.ds(0, CHUNK0)], sem),
            pltpu.make_async_copy(
                table_ref.at[idx_v.at[pl.ds(off + CHUNK0, CHUNK1)]],
                buf.at[pl.ds(CHUNK0, CHUNK1)], sem),
        )

    def issue(g, slot):
        for d in gather_descrs(g, slot):
            d.start()

    def drain(g, slot):
        for d in gather_descrs(g, slot):
            d.wait()

    def compute(g, slot):
        buf = bufs[slot]
        for bag in range(GROUP_BAGS):
            def rbody(r, acc):
                row = bag * L + r
                return tuple(acc[k] + buf[row, pl.ds(k * 16, 16)]
                             for k in range(VREGS))
            acc = lax.fori_loop(
                0, L, rbody,
                tuple(jnp.zeros((16,), jnp.float32) for _ in range(VREGS)))
            for k in range(VREGS):
                pooled_v[g * GROUP_BAGS + bag, pl.ds(k * 16, 16)] = acc[k]

    issue(0, 0)

    def outer(g2, carry):
        for b in range(2):
            g = g2 * 2 + b

            @pl.when(g + 1 < NGROUPS)
            def _():
                issue(g + 1, 1 - b)

            drain(g, b)
            compute(g, b)
        return carry

    lax.fori_loop(0, NGROUPS // 2, outer, 0)

    pltpu.sync_copy(pooled_v, out_ref.at[pl.ds(wid * BAGS_PER_W, BAGS_PER_W)])


def _tc_head(pooled_ref, w_ref, b_ref, out_ref):
    out_ref[...] = (
        jnp.dot(pooled_ref[...], w_ref[...],
                preferred_element_type=jnp.float32)
        + b_ref[...]
    )


@jax.jit
def kernel(data, table, W, b):
    data_flat = data.reshape(-1)

    sc_pool = pl.kernel(
        _sc_body,
        out_type=jax.ShapeDtypeStruct((B, EMBED), jnp.float32),
        mesh=plsc.VectorSubcoreMesh(
            core_axis_name="c", subcore_axis_name="s",
            num_cores=NC, num_subcores=NS),
        scratch_types=[
            pltpu.VMEM((IDX_PER_W,), jnp.int32),
            pltpu.VMEM((GROUP_IDX, EMBED), jnp.float32),
            pltpu.VMEM((GROUP_IDX, EMBED), jnp.float32),
            pltpu.VMEM((BAGS_PER_W, EMBED), jnp.float32),
            pltpu.SemaphoreType.DMA,
            pltpu.SemaphoreType.DMA,
        ],
        compiler_params=pltpu.CompilerParams(use_tc_tiling_on_sc=False),
    )
    pooled_sum = sc_pool(data_flat, table)

    w_pad = jnp.zeros((EMBED, 128), jnp.float32)
    w_pad = lax.dynamic_update_slice(w_pad, W.T * (1.0 / L), (0, 0))
    b_pad = jnp.zeros((1, 128), jnp.float32)
    b_pad = lax.dynamic_update_slice(b_pad, b[None, :], (0, 0))

    blk = 2048
    logits_pad = pl.pallas_call(
        _tc_head,
        grid=(B // blk,),
        in_specs=[
            pl.BlockSpec((blk, EMBED), lambda i: (i, 0)),
            pl.BlockSpec((EMBED, 128), lambda i: (0, 0)),
            pl.BlockSpec((1, 128), lambda i: (0, 0)),
        ],
        out_specs=pl.BlockSpec((blk, 128), lambda i: (i, 0)),
        out_shape=jax.ShapeDtypeStruct((B, 128), jnp.float32),
    )(pooled_sum, w_pad, b_pad)

    return logits_pad[:, :NUM_CLASSES]
